# baseline (speedup 1.0000x reference)
import functools

import jax
import jax.numpy as jnp
from jax import lax
from jax.experimental import pallas as pl
from jax.experimental.pallas import tpu as pltpu
from jax.experimental.pallas import tpu_sc as plsc

DIM = 4096
N_EXPERTS = 64
TOPK = 8
N_GROUPS = 8
GROUP_SIZE = N_EXPERTS // N_GROUPS
TOPK_GROUPS = 4
ROUTE_SCALE = 2.5
N_TOK = 8192

BLOCK_T = 1024
NW = 32
TPW = N_TOK // NW
SLABS = TPW // 16


def _dense(x, wT, b2, tpw):
    n = x.shape[0]
    nblk = n // tpw
    bt = min(BLOCK_T, n)

    def _dense_block(x_ref, wT_ref, b_ref, s_out_ref, o_out_ref):
        xv = x_ref[...]
        wTv = wT_ref[...]
        logits = jnp.dot(xv, wTv, preferred_element_type=jnp.float32)
        lt = logits.T
        origT = jax.nn.sigmoid(lt)
        sT = origT + b_ref[...]
        for q in range(bt // tpw):
            s_out_ref[q] = sT[:, q * tpw:(q + 1) * tpw]
            o_out_ref[q] = origT[:, q * tpw:(q + 1) * tpw]

    return pl.pallas_call(
        _dense_block,
        grid=(n // bt,),
        in_specs=[
            pl.BlockSpec((bt, DIM), lambda i: (i, 0)),
            pl.BlockSpec((DIM, N_EXPERTS), lambda i: (0, 0)),
            pl.BlockSpec((N_EXPERTS, 1), lambda i: (0, 0)),
        ],
        out_specs=[
            pl.BlockSpec((bt // tpw, N_EXPERTS, tpw), lambda i: (i, 0, 0)),
            pl.BlockSpec((bt // tpw, N_EXPERTS, tpw), lambda i: (i, 0, 0)),
        ],
        out_shape=[
            jax.ShapeDtypeStruct((nblk, N_EXPERTS, tpw), jnp.float32),
            jax.ShapeDtypeStruct((nblk, N_EXPERTS, tpw), jnp.float32),
        ],
        compiler_params=pltpu.CompilerParams(
            dimension_semantics=("arbitrary",),
        ),
    )(x, wT, b2)


def _make_route_body(tpw):
  def _route_body(s_hbm, o_hbm, w_hbm, i_hbm, s_v, o_v, vt, it, wo, io):
    w = lax.axis_index("s") * 2 + lax.axis_index("c")
    pltpu.sync_copy(s_hbm.at[w], s_v)
    pltpu.sync_copy(o_hbm.at[w], o_v)
    lanes = lax.iota(jnp.int32, 16)

    for e in range(N_EXPERTS):
        it[pl.ds((N_EXPERTS + e) * 16, 16)] = jnp.full((16,), e, jnp.int32)

    def slab(j, carry):
        t0 = j * 16

        gkey = []
        for g in range(N_GROUPS):
            v0 = s_v[pl.ds(g * GROUP_SIZE * tpw + t0, 16)]
            v1 = s_v[pl.ds((g * GROUP_SIZE + 1) * tpw + t0, 16)]
            m1 = jnp.maximum(v0, v1)
            m2 = jnp.minimum(v0, v1)
            for e in range(2, GROUP_SIZE):
                v = s_v[pl.ds((g * GROUP_SIZE + e) * tpw + t0, 16)]
                m2 = jnp.maximum(m2, jnp.minimum(m1, v))
                m1 = jnp.maximum(m1, v)
            gkey.append(m1 + m2)

        gval = [None] * 16
        gidx = [None] * 16
        for g in range(N_GROUPS):
            gval[8 + g] = gkey[g]
            gidx[8 + g] = jnp.full((16,), g, jnp.int32)
        for nd in range(7, 0, -1):
            take = gval[2 * nd] >= gval[2 * nd + 1]
            gval[nd] = jnp.where(take, gval[2 * nd], gval[2 * nd + 1])
            gidx[nd] = jnp.where(take, gidx[2 * nd], gidx[2 * nd + 1])
        for _ in range(TOPK_GROUPS):
            gstar = gidx[1]
            for g in range(N_GROUPS):
                rm = gstar == g
                gval[8 + g] = jnp.where(rm, -jnp.inf, gval[8 + g])
            for nd in range(7, 0, -1):
                take = gval[2 * nd] >= gval[2 * nd + 1]
                gval[nd] = jnp.where(take, gval[2 * nd], gval[2 * nd + 1])
                gidx[nd] = jnp.where(take, gidx[2 * nd], gidx[2 * nd + 1])
        fl = [gval[8 + g] == -jnp.inf for g in range(N_GROUPS)]

        for e in range(N_EXPERTS):
            v = s_v[pl.ds(e * tpw + t0, 16)]
            leaf = jnp.where(fl[e // GROUP_SIZE], v, -jnp.inf)
            vt[pl.ds((N_EXPERTS + e) * 16, 16)] = leaf
        for nd in range(N_EXPERTS - 1, 0, -1):
            a = vt[pl.ds(2 * nd * 16, 16)]
            b = vt[pl.ds((2 * nd + 1) * 16, 16)]
            ia = it[pl.ds(2 * nd * 16, 16)]
            ib = it[pl.ds((2 * nd + 1) * 16, 16)]
            take = a >= b
            vt[pl.ds(nd * 16, 16)] = jnp.where(take, a, b)
            it[pl.ds(nd * 16, 16)] = jnp.where(take, ia, ib)

        wks = []
        wsum = jnp.zeros((16,), jnp.float32)
        for k in range(TOPK):
            estar = it[pl.ds(16, 16)]
            io[pl.ds(k * tpw + t0, 16)] = estar
            wk = plsc.load_gather(o_v, [estar * tpw + t0 + lanes])
            wks.append(wk)
            wsum = wsum + wk
            if k < TOPK - 1:
                la = (estar + N_EXPERTS) * 16 + lanes
                plsc.store_scatter(vt, [la], jnp.full((16,), -jnp.inf, jnp.float32))
                node = lax.shift_right_logical(estar + N_EXPERTS, 1)
                for _ in range(6):
                    c0a = node * 32 + lanes
                    c1a = node * 32 + 16 + lanes
                    av = plsc.load_gather(vt, [c0a])
                    bv = plsc.load_gather(vt, [c1a])
                    ai = plsc.load_gather(it, [c0a])
                    bi = plsc.load_gather(it, [c1a])
                    take = av >= bv
                    na = node * 16 + lanes
                    plsc.store_scatter(vt, [na], jnp.where(take, av, bv))
                    plsc.store_scatter(it, [na], jnp.where(take, ai, bi))
                    node = lax.shift_right_logical(node, 1)
        for k in range(TOPK):
            wo[pl.ds(k * tpw + t0, 16)] = (wks[k] / wsum) * ROUTE_SCALE
        return carry

    lax.fori_loop(0, tpw // 16, slab, 0)
    pltpu.sync_copy(wo, w_hbm.at[w])
    pltpu.sync_copy(io, i_hbm.at[w])
  return _route_body


def _route(s3, o3, tpw):
    mesh = plsc.VectorSubcoreMesh(core_axis_name="c", subcore_axis_name="s")
    call = pl.kernel(
        _make_route_body(tpw),
        out_type=[
            jax.ShapeDtypeStruct((NW, TOPK * tpw), jnp.float32),
            jax.ShapeDtypeStruct((NW, TOPK * tpw), jnp.int32),
        ],
        mesh=mesh,
        scratch_types=[
            pltpu.VMEM((N_EXPERTS * tpw,), jnp.float32),
            pltpu.VMEM((N_EXPERTS * tpw,), jnp.float32),
            pltpu.VMEM((2 * N_EXPERTS * 16,), jnp.float32),
            pltpu.VMEM((2 * N_EXPERTS * 16,), jnp.int32),
            pltpu.VMEM((TOPK * tpw,), jnp.float32),
            pltpu.VMEM((TOPK * tpw,), jnp.int32),
        ],
        compiler_params=pltpu.CompilerParams(needs_layout_passes=False),
    )
    return call(s3, o3)


CHUNKS = 4


def kernel(x, weight, bias):
    n = x.shape[0]
    wT = weight.T
    b2 = bias.reshape(N_EXPERTS, 1)
    ctok = n // CHUNKS
    tpw = ctok // NW
    w_parts = []
    i_parts = []
    for c in range(CHUNKS):
        xc = lax.slice_in_dim(x, c * ctok, (c + 1) * ctok, axis=0)
        s3, o3 = _dense(xc, wT, b2, tpw)
        nblk = ctok // tpw
        w3, i3 = _route(s3.reshape(nblk, N_EXPERTS * tpw),
                        o3.reshape(nblk, N_EXPERTS * tpw), tpw)
        w_parts.append(
            w3.reshape(nblk, TOPK, tpw).transpose(0, 2, 1).reshape(ctok, TOPK))
        i_parts.append(
            i3.reshape(nblk, TOPK, tpw).transpose(0, 2, 1).reshape(ctok, TOPK))
    return (jnp.concatenate(w_parts, axis=0),
            jnp.concatenate(i_parts, axis=0))


# speedup vs baseline: 2.0977x; 2.0977x over previous
import functools

import jax
import jax.numpy as jnp
from jax import lax
from jax.experimental import pallas as pl
from jax.experimental.pallas import tpu as pltpu
from jax.experimental.pallas import tpu_sc as plsc

DIM = 4096
N_EXPERTS = 64
TOPK = 8
N_GROUPS = 8
GROUP_SIZE = N_EXPERTS // N_GROUPS
TOPK_GROUPS = 4
ROUTE_SCALE = 2.5
N_TOK = 8192

BLOCK_T = 1024
NW = 32
TPW = N_TOK // NW
SLABS = TPW // 16


def _dense(x, wT, b2, tpw):
    n = x.shape[0]
    nblk = n // tpw
    bt = min(BLOCK_T, n)

    def _dense_block(x_ref, wT_ref, b_ref, s_out_ref, o_out_ref):
        xv = x_ref[...]
        wTv = wT_ref[...]
        logits = jnp.dot(xv, wTv, preferred_element_type=jnp.float32)
        lt = logits.T
        origT = jax.nn.sigmoid(lt)
        sT = origT + b_ref[...]
        for q in range(bt // tpw):
            s_out_ref[q] = sT[:, q * tpw:(q + 1) * tpw]
            o_out_ref[q] = origT[:, q * tpw:(q + 1) * tpw]

    return pl.pallas_call(
        _dense_block,
        grid=(n // bt,),
        in_specs=[
            pl.BlockSpec((bt, DIM), lambda i: (i, 0)),
            pl.BlockSpec((DIM, N_EXPERTS), lambda i: (0, 0)),
            pl.BlockSpec((N_EXPERTS, 1), lambda i: (0, 0)),
        ],
        out_specs=[
            pl.BlockSpec((bt // tpw, N_EXPERTS, tpw), lambda i: (i, 0, 0)),
            pl.BlockSpec((bt // tpw, N_EXPERTS, tpw), lambda i: (i, 0, 0)),
        ],
        out_shape=[
            jax.ShapeDtypeStruct((nblk, N_EXPERTS, tpw), jnp.float32),
            jax.ShapeDtypeStruct((nblk, N_EXPERTS, tpw), jnp.float32),
        ],
        compiler_params=pltpu.CompilerParams(
            dimension_semantics=("arbitrary",),
        ),
    )(x, wT, b2)


def _make_route_body(tpw):
  def _route_body(s_hbm, o_hbm, w_hbm, i_hbm, s_v, o_v, vt, it, wo, io):
    w = lax.axis_index("s") * 2 + lax.axis_index("c")
    pltpu.sync_copy(s_hbm.at[w], s_v)
    pltpu.sync_copy(o_hbm.at[w], o_v)
    lanes = lax.iota(jnp.int32, 16)

    for e in range(N_EXPERTS):
        it[pl.ds((N_EXPERTS + e) * 16, 16)] = jnp.full((16,), e, jnp.int32)

    def slab(j, carry):
        t0 = j * 16

        gkey = []
        for g in range(N_GROUPS):
            v0 = s_v[pl.ds(g * GROUP_SIZE * tpw + t0, 16)]
            v1 = s_v[pl.ds((g * GROUP_SIZE + 1) * tpw + t0, 16)]
            m1 = jnp.maximum(v0, v1)
            m2 = jnp.minimum(v0, v1)
            for e in range(2, GROUP_SIZE):
                v = s_v[pl.ds((g * GROUP_SIZE + e) * tpw + t0, 16)]
                m2 = jnp.maximum(m2, jnp.minimum(m1, v))
                m1 = jnp.maximum(m1, v)
            gkey.append(m1 + m2)

        gval = [None] * 16
        gidx = [None] * 16
        for g in range(N_GROUPS):
            gval[8 + g] = gkey[g]
            gidx[8 + g] = jnp.full((16,), g, jnp.int32)
        for nd in range(7, 0, -1):
            take = gval[2 * nd] >= gval[2 * nd + 1]
            gval[nd] = jnp.where(take, gval[2 * nd], gval[2 * nd + 1])
            gidx[nd] = jnp.where(take, gidx[2 * nd], gidx[2 * nd + 1])
        for _ in range(TOPK_GROUPS):
            gstar = gidx[1]
            for g in range(N_GROUPS):
                rm = gstar == g
                gval[8 + g] = jnp.where(rm, -jnp.inf, gval[8 + g])
            for nd in range(7, 0, -1):
                take = gval[2 * nd] >= gval[2 * nd + 1]
                gval[nd] = jnp.where(take, gval[2 * nd], gval[2 * nd + 1])
                gidx[nd] = jnp.where(take, gidx[2 * nd], gidx[2 * nd + 1])
        fl = [gval[8 + g] == -jnp.inf for g in range(N_GROUPS)]

        for e in range(N_EXPERTS):
            v = s_v[pl.ds(e * tpw + t0, 16)]
            leaf = jnp.where(fl[e // GROUP_SIZE], v, -jnp.inf)
            vt[pl.ds((N_EXPERTS + e) * 16, 16)] = leaf
        for nd in range(N_EXPERTS - 1, 0, -1):
            a = vt[pl.ds(2 * nd * 16, 16)]
            b = vt[pl.ds((2 * nd + 1) * 16, 16)]
            ia = it[pl.ds(2 * nd * 16, 16)]
            ib = it[pl.ds((2 * nd + 1) * 16, 16)]
            take = a >= b
            vt[pl.ds(nd * 16, 16)] = jnp.where(take, a, b)
            it[pl.ds(nd * 16, 16)] = jnp.where(take, ia, ib)

        wks = []
        wsum = jnp.zeros((16,), jnp.float32)
        for k in range(TOPK):
            estar = it[pl.ds(16, 16)]
            io[pl.ds(k * tpw + t0, 16)] = estar
            wk = plsc.load_gather(o_v, [estar * tpw + t0 + lanes])
            wks.append(wk)
            wsum = wsum + wk
            if k < TOPK - 1:
                la = (estar + N_EXPERTS) * 16 + lanes
                plsc.store_scatter(vt, [la], jnp.full((16,), -jnp.inf, jnp.float32))
                node = lax.shift_right_logical(estar + N_EXPERTS, 1)
                for _ in range(6):
                    c0a = node * 32 + lanes
                    c1a = node * 32 + 16 + lanes
                    av = plsc.load_gather(vt, [c0a])
                    bv = plsc.load_gather(vt, [c1a])
                    ai = plsc.load_gather(it, [c0a])
                    bi = plsc.load_gather(it, [c1a])
                    take = av >= bv
                    na = node * 16 + lanes
                    plsc.store_scatter(vt, [na], jnp.where(take, av, bv))
                    plsc.store_scatter(it, [na], jnp.where(take, ai, bi))
                    node = lax.shift_right_logical(node, 1)
        for k in range(TOPK):
            wo[pl.ds(k * tpw + t0, 16)] = (wks[k] / wsum) * ROUTE_SCALE
        return carry

    lax.fori_loop(0, tpw // 16, slab, 0)
    pltpu.sync_copy(wo, w_hbm.at[w])
    pltpu.sync_copy(io, i_hbm.at[w])
  return _route_body


def _route(s3, o3, tpw):
    mesh = plsc.VectorSubcoreMesh(core_axis_name="c", subcore_axis_name="s")
    call = pl.kernel(
        _make_route_body(tpw),
        out_type=[
            jax.ShapeDtypeStruct((NW, TOPK * tpw), jnp.float32),
            jax.ShapeDtypeStruct((NW, TOPK * tpw), jnp.int32),
        ],
        mesh=mesh,
        scratch_types=[
            pltpu.VMEM((N_EXPERTS * tpw,), jnp.float32),
            pltpu.VMEM((N_EXPERTS * tpw,), jnp.float32),
            pltpu.VMEM((2 * N_EXPERTS * 16,), jnp.float32),
            pltpu.VMEM((2 * N_EXPERTS * 16,), jnp.int32),
            pltpu.VMEM((TOPK * tpw,), jnp.float32),
            pltpu.VMEM((TOPK * tpw,), jnp.int32),
        ],
        compiler_params=pltpu.CompilerParams(needs_layout_passes=False),
    )
    return call(s3, o3)


CHUNKS = 1


def kernel(x, weight, bias):
    n = x.shape[0]
    wT = weight.T
    b2 = bias.reshape(N_EXPERTS, 1)
    ctok = n // CHUNKS
    tpw = ctok // NW
    w_parts = []
    i_parts = []
    for c in range(CHUNKS):
        xc = lax.slice_in_dim(x, c * ctok, (c + 1) * ctok, axis=0)
        s3, o3 = _dense(xc, wT, b2, tpw)
        nblk = ctok // tpw
        w3, i3 = _route(s3.reshape(nblk, N_EXPERTS * tpw),
                        o3.reshape(nblk, N_EXPERTS * tpw), tpw)
        w_parts.append(
            w3.reshape(nblk, TOPK, tpw).transpose(0, 2, 1).reshape(ctok, TOPK))
        i_parts.append(
            i3.reshape(nblk, TOPK, tpw).transpose(0, 2, 1).reshape(ctok, TOPK))
    return (jnp.concatenate(w_parts, axis=0),
            jnp.concatenate(i_parts, axis=0))


# speedup vs baseline: 2.0984x; 1.0003x over previous
import jax
import jax.numpy as jnp
from jax import lax
from jax.experimental import pallas as pl
from jax.experimental.pallas import tpu as pltpu
from jax.experimental.pallas import tpu_sc as plsc

DIM = 4096
N_EXPERTS = 64
TOPK = 8
N_GROUPS = 8
GROUP_SIZE = N_EXPERTS // N_GROUPS
TOPK_GROUPS = 4
ROUTE_SCALE = 2.5
N_TOK = 8192

BLOCK_T = 1024
NW = 32
TPW = N_TOK // NW


def _dense(x, wT, b2, tpw):
    n = x.shape[0]
    nblk = n // tpw
    bt = min(BLOCK_T, n)

    def _dense_block(x_ref, wT_ref, b_ref, s_out_ref, o_out_ref):
        xv = x_ref[...]
        wTv = wT_ref[...]
        logits = jnp.dot(xv, wTv, preferred_element_type=jnp.float32)
        lt = logits.T
        origT = jax.nn.sigmoid(lt)
        sT = origT + b_ref[...]
        for q in range(bt // tpw):
            s_out_ref[q] = sT[:, q * tpw:(q + 1) * tpw]
            o_out_ref[q] = origT[:, q * tpw:(q + 1) * tpw]

    return pl.pallas_call(
        _dense_block,
        grid=(n // bt,),
        in_specs=[
            pl.BlockSpec((bt, DIM), lambda i: (i, 0)),
            pl.BlockSpec((DIM, N_EXPERTS), lambda i: (0, 0)),
            pl.BlockSpec((N_EXPERTS, 1), lambda i: (0, 0)),
        ],
        out_specs=[
            pl.BlockSpec((bt // tpw, N_EXPERTS, tpw), lambda i: (i, 0, 0)),
            pl.BlockSpec((bt // tpw, N_EXPERTS, tpw), lambda i: (i, 0, 0)),
        ],
        out_shape=[
            jax.ShapeDtypeStruct((nblk, N_EXPERTS, tpw), jnp.float32),
            jax.ShapeDtypeStruct((nblk, N_EXPERTS, tpw), jnp.float32),
        ],
        compiler_params=pltpu.CompilerParams(
            dimension_semantics=("arbitrary",),
        ),
    )(x, wT, b2)


def _make_route_body(tpw):
  def _route_body(s_hbm, o_hbm, w_hbm, i_hbm, s_v, o_v, vt, it, wo, io):
    w = lax.axis_index("s") * 2 + lax.axis_index("c")
    pltpu.sync_copy(s_hbm.at[w], s_v)
    pltpu.sync_copy(o_hbm.at[w], o_v)
    lanes = lax.iota(jnp.int32, 16)

    for e in range(N_EXPERTS):
        it[pl.ds((N_EXPERTS + e) * 16, 16)] = jnp.full((16,), e, jnp.int32)

    def slab(j, carry):
        t0 = j * 16

        gkey = []
        for g in range(N_GROUPS):
            v0 = s_v[pl.ds(g * GROUP_SIZE * tpw + t0, 16)]
            v1 = s_v[pl.ds((g * GROUP_SIZE + 1) * tpw + t0, 16)]
            m1 = jnp.maximum(v0, v1)
            m2 = jnp.minimum(v0, v1)
            for e in range(2, GROUP_SIZE):
                v = s_v[pl.ds((g * GROUP_SIZE + e) * tpw + t0, 16)]
                m2 = jnp.maximum(m2, jnp.minimum(m1, v))
                m1 = jnp.maximum(m1, v)
            gkey.append(m1 + m2)

        gval = [None] * 16
        gidx = [None] * 16
        for g in range(N_GROUPS):
            gval[8 + g] = gkey[g]
            gidx[8 + g] = jnp.full((16,), g, jnp.int32)
        for nd in range(7, 0, -1):
            take = gval[2 * nd] >= gval[2 * nd + 1]
            gval[nd] = jnp.where(take, gval[2 * nd], gval[2 * nd + 1])
            gidx[nd] = jnp.where(take, gidx[2 * nd], gidx[2 * nd + 1])
        for _ in range(TOPK_GROUPS):
            gstar = gidx[1]
            for g in range(N_GROUPS):
                rm = gstar == g
                gval[8 + g] = jnp.where(rm, -jnp.inf, gval[8 + g])
            for nd in range(7, 0, -1):
                take = gval[2 * nd] >= gval[2 * nd + 1]
                gval[nd] = jnp.where(take, gval[2 * nd], gval[2 * nd + 1])
                gidx[nd] = jnp.where(take, gidx[2 * nd], gidx[2 * nd + 1])
        fl = [gval[8 + g] == -jnp.inf for g in range(N_GROUPS)]

        for e in range(N_EXPERTS):
            v = s_v[pl.ds(e * tpw + t0, 16)]
            leaf = jnp.where(fl[e // GROUP_SIZE], v, -jnp.inf)
            vt[pl.ds((N_EXPERTS + e) * 16, 16)] = leaf
        for nd in range(N_EXPERTS - 1, 0, -1):
            a = vt[pl.ds(2 * nd * 16, 16)]
            b = vt[pl.ds((2 * nd + 1) * 16, 16)]
            ia = it[pl.ds(2 * nd * 16, 16)]
            ib = it[pl.ds((2 * nd + 1) * 16, 16)]
            take = a >= b
            vt[pl.ds(nd * 16, 16)] = jnp.where(take, a, b)
            it[pl.ds(nd * 16, 16)] = jnp.where(take, ia, ib)

        wks = []
        wsum = jnp.zeros((16,), jnp.float32)
        for k in range(TOPK):
            estar = it[pl.ds(16, 16)]
            io[pl.ds(k * tpw + t0, 16)] = estar
            wk = plsc.load_gather(o_v, [estar * tpw + t0 + lanes])
            wks.append(wk)
            wsum = wsum + wk
            if k < TOPK - 1:
                la = (estar + N_EXPERTS) * 16 + lanes
                plsc.store_scatter(vt, [la], jnp.full((16,), -jnp.inf, jnp.float32))
                node = lax.shift_right_logical(estar + N_EXPERTS, 1)
                for _ in range(6):
                    c0a = node * 32 + lanes
                    c1a = node * 32 + 16 + lanes
                    av = plsc.load_gather(vt, [c0a])
                    bv = plsc.load_gather(vt, [c1a])
                    ai = plsc.load_gather(it, [c0a])
                    bi = plsc.load_gather(it, [c1a])
                    take = av >= bv
                    na = node * 16 + lanes
                    plsc.store_scatter(vt, [na], jnp.where(take, av, bv))
                    plsc.store_scatter(it, [na], jnp.where(take, ai, bi))
                    node = lax.shift_right_logical(node, 1)
        for k in range(TOPK):
            wo[pl.ds(k * tpw + t0, 16)] = (wks[k] / wsum) * ROUTE_SCALE
        return carry

    lax.fori_loop(0, tpw // 16, slab, 0)
    pltpu.sync_copy(wo, w_hbm.at[w])
    pltpu.sync_copy(io, i_hbm.at[w])
  return _route_body


def _route(s3, o3, tpw):
    mesh = plsc.VectorSubcoreMesh(core_axis_name="c", subcore_axis_name="s")
    call = pl.kernel(
        _make_route_body(tpw),
        out_type=[
            jax.ShapeDtypeStruct((NW, TOPK * tpw), jnp.float32),
            jax.ShapeDtypeStruct((NW, TOPK * tpw), jnp.int32),
        ],
        mesh=mesh,
        scratch_types=[
            pltpu.VMEM((N_EXPERTS * tpw,), jnp.float32),
            pltpu.VMEM((N_EXPERTS * tpw,), jnp.float32),
            pltpu.VMEM((2 * N_EXPERTS * 16,), jnp.float32),
            pltpu.VMEM((2 * N_EXPERTS * 16,), jnp.int32),
            pltpu.VMEM((TOPK * tpw,), jnp.float32),
            pltpu.VMEM((TOPK * tpw,), jnp.int32),
        ],
        compiler_params=pltpu.CompilerParams(needs_layout_passes=False),
    )
    return call(s3, o3)


def kernel(x, weight, bias):
    n = x.shape[0]
    wT = weight.T
    b2 = bias.reshape(N_EXPERTS, 1)
    tpw = n // NW
    s3, o3 = _dense(x, wT, b2, tpw)
    w3, i3 = _route(s3.reshape(NW, N_EXPERTS * tpw),
                    o3.reshape(NW, N_EXPERTS * tpw), tpw)
    w_out = w3.reshape(NW, TOPK, tpw).transpose(0, 2, 1).reshape(n, TOPK)
    idx_out = i3.reshape(NW, TOPK, tpw).transpose(0, 2, 1).reshape(n, TOPK)
    return w_out, idx_out
